# Initial kernel scaffold; baseline (speedup 1.0000x reference)
import functools

import jax
import jax.numpy as jnp
from jax import lax
from jax.experimental import pallas as pl
from jax.experimental.pallas import tpu as pltpu
from jax.experimental.pallas import tpu_sc as plsc

NC = 2
NS = 16
NW = NC * NS
CH = 128
GPT = 80
EPT = CH * GPT
EP = NW * EPT
NP = 10240
STRIPE = NP // NS
D = 128
ROWB = 512
GRID = NP // ROWB

_mesh = plsc.VectorSubcoreMesh(
    core_axis_name="c", subcore_axis_name="s", num_cores=NC, num_subcores=NS
)


def _count_body(dst_ref, out_ref, idx_v, cnt_v):
    cid = lax.axis_index("c")
    sid = lax.axis_index("s")
    wid = cid * NS + sid
    pltpu.sync_copy(dst_ref.at[wid], idx_v)
    zeros = jnp.zeros((16,), jnp.float32)

    def zero_body(i, carry):
        cnt_v[pl.ds(i * 16, 16)] = zeros
        return carry

    lax.fori_loop(0, NP // 16, zero_body, 0)
    ones = jnp.full((16,), 1.0, jnp.float32)

    def scat_body(i, carry):
        idx = idx_v[pl.ds(i * 16, 16)]
        plsc.addupdate_scatter(cnt_v, [idx], ones)
        return carry

    lax.fori_loop(0, EPT // 16, scat_body, 0)
    pltpu.sync_copy(cnt_v, out_ref.at[wid])


_count_kernel = functools.partial(
    pl.kernel,
    out_type=jax.ShapeDtypeStruct((NW, NP), jnp.float32),
    mesh=_mesh,
    scratch_types=[
        pltpu.VMEM((EPT,), jnp.int32),
        pltpu.VMEM((NP,), jnp.float32),
    ],
)(_count_body)


def _mm_body(x_ref, w_ref, cnt_ref, h2_ref):
    cnt = jnp.sum(cnt_ref[...], axis=0)
    dinv = lax.rsqrt(cnt + 1.0)
    h = jnp.dot(x_ref[...], w_ref[...], preferred_element_type=jnp.float32)
    h2_ref[...] = h * dinv[:, None]


def _agg_body(h2_ref, srcg_ref, dstg_ref, out_ref,
              acc_sh, idxs_v, idxd_v, buf0, buf1, sem0, sem1):
    cid = lax.axis_index("c")
    sid = lax.axis_index("s")
    wid = cid * NS + sid
    pltpu.sync_copy(srcg_ref.at[wid], idxs_v)
    pltpu.sync_copy(dstg_ref.at[wid], idxd_v)

    zeros = jnp.zeros((16,), jnp.float32)

    def zero_body(r, carry):
        for c in range(D // 16):
            buf0[r, pl.ds(c * 16, 16)] = zeros
        return carry

    lax.fori_loop(0, CH, zero_body, 0)
    base = sid * STRIPE
    for z in range(STRIPE // CH):
        pltpu.sync_copy(buf0, acc_sh.at[pl.ds(base + z * CH, CH)])
    plsc.subcore_barrier()

    pltpu.make_async_copy(h2_ref.at[idxs_v.at[0]], buf0, sem0).start()
    pltpu.make_async_copy(h2_ref.at[idxs_v.at[1]], buf1, sem1).start()

    def loop(g, carry):
        j0 = g * 2
        for bi, (buf, sem) in enumerate(((buf0, sem0), (buf1, sem1))):
            j = j0 + bi
            pltpu.make_async_copy(h2_ref.at[idxs_v.at[j]], buf, sem).wait()
            pltpu.sync_copy(buf, acc_sh.at[idxd_v.at[j]], add=True)
            pltpu.make_async_copy(h2_ref.at[idxs_v.at[j + 2]], buf, sem).start()
        return carry

    lax.fori_loop(0, GPT // 2, loop, 0)
    pltpu.make_async_copy(h2_ref.at[idxs_v.at[GPT]], buf0, sem0).wait()
    pltpu.make_async_copy(h2_ref.at[idxs_v.at[GPT + 1]], buf1, sem1).wait()
    plsc.subcore_barrier()

    for z in range(STRIPE // CH):
        pltpu.sync_copy(acc_sh.at[pl.ds(base + z * CH, CH)], buf0)
        pltpu.sync_copy(buf0, out_ref.at[cid, pl.ds(base + z * CH, CH)])


_agg_kernel = functools.partial(
    pl.kernel,
    out_type=jax.ShapeDtypeStruct((NC, NP, D), jnp.float32),
    mesh=_mesh,
    scratch_types=[
        pltpu.VMEM_SHARED((NP, D), jnp.float32),
        pltpu.VMEM((GPT + 2, CH), jnp.int32),
        pltpu.VMEM((GPT, CH), jnp.int32),
        pltpu.VMEM((CH, D), jnp.float32),
        pltpu.VMEM((CH, D), jnp.float32),
        pltpu.SemaphoreType.DMA,
        pltpu.SemaphoreType.DMA,
    ],
)(_agg_body)


def _fin_body(acc_ref, h2_ref, cnt_ref, b_ref, o_ref):
    cnt = jnp.sum(cnt_ref[...], axis=0)
    dinv = lax.rsqrt(cnt + 1.0)[:, None]
    s = acc_ref[0] + acc_ref[1] + h2_ref[...]
    o_ref[...] = jnp.maximum(s * dinv + b_ref[...], 0.0)


def kernel(x, edge_index, W, b):
    N = x.shape[0]
    E = edge_index.shape[1]
    ei = edge_index.astype(jnp.int32)
    pad = jnp.full((EP - E,), NP - 1, jnp.int32)
    src = jnp.concatenate([ei[0], pad])
    dst = jnp.concatenate([ei[1], pad])
    srcg = src.reshape(NW, GPT, CH)
    srcg = jnp.concatenate([srcg, jnp.zeros((NW, 2, CH), jnp.int32)], axis=1)
    dstg = dst.reshape(NW, GPT, CH)
    dst_flat = dst.reshape(NW, EPT)
    x_p = jnp.pad(x, ((0, NP - N), (0, 0)))

    cnt_part = _count_kernel(dst_flat)

    h2 = pl.pallas_call(
        _mm_body,
        grid=(GRID,),
        in_specs=[
            pl.BlockSpec((ROWB, D), lambda i: (i, 0)),
            pl.BlockSpec((D, D), lambda i: (0, 0)),
            pl.BlockSpec((NW, ROWB), lambda i: (0, i)),
        ],
        out_specs=pl.BlockSpec((ROWB, D), lambda i: (i, 0)),
        out_shape=jax.ShapeDtypeStruct((NP, D), jnp.float32),
    )(x_p, W, cnt_part)

    accp = _agg_kernel(h2, srcg, dstg)

    out_p = pl.pallas_call(
        _fin_body,
        grid=(GRID,),
        in_specs=[
            pl.BlockSpec((NC, ROWB, D), lambda i: (0, i, 0)),
            pl.BlockSpec((ROWB, D), lambda i: (i, 0)),
            pl.BlockSpec((NW, ROWB), lambda i: (0, i)),
            pl.BlockSpec((1, D), lambda i: (0, 0)),
        ],
        out_specs=pl.BlockSpec((ROWB, D), lambda i: (i, 0)),
        out_shape=jax.ShapeDtypeStruct((NP, D), jnp.float32),
    )(accp, h2, cnt_part, b.reshape(1, D))

    return out_p[:N]


# speedup vs baseline: 11.0816x; 11.0816x over previous
import functools

import jax
import jax.numpy as jnp
from jax import lax
from jax.experimental import pallas as pl
from jax.experimental.pallas import tpu as pltpu
from jax.experimental.pallas import tpu_sc as plsc

NC = 2
NS = 16
NW = NC * NS
CH = 128
GPT = 80
EPT = CH * GPT
EP = NW * EPT
NP = 10240
STRIPE = NP // NS
D = 128
ROWB = 512
GRID = NP // ROWB

_mesh = plsc.VectorSubcoreMesh(
    core_axis_name="c", subcore_axis_name="s", num_cores=NC, num_subcores=NS
)


def _count_body(dst_ref, out_ref, idx_v, cnt_v):
    cid = lax.axis_index("c")
    sid = lax.axis_index("s")
    wid = cid * NS + sid
    pltpu.sync_copy(dst_ref.at[wid], idx_v)
    zeros = jnp.zeros((16,), jnp.float32)

    def zero_body(i, carry):
        cnt_v[pl.ds(i * 16, 16)] = zeros
        return carry

    lax.fori_loop(0, NP // 16, zero_body, 0)
    ones = jnp.full((16,), 1.0, jnp.float32)

    def scat_body(i, carry):
        idx = idx_v[pl.ds(i * 16, 16)]
        plsc.addupdate_scatter(cnt_v, [idx], ones)
        return carry

    lax.fori_loop(0, EPT // 16, scat_body, 0)
    pltpu.sync_copy(cnt_v, out_ref.at[wid])


_count_kernel = functools.partial(
    pl.kernel,
    out_type=jax.ShapeDtypeStruct((NW, NP), jnp.float32),
    mesh=_mesh,
    compiler_params=pltpu.CompilerParams(needs_layout_passes=False),
    scratch_types=[
        pltpu.VMEM((EPT,), jnp.int32),
        pltpu.VMEM((NP,), jnp.float32),
    ],
)(_count_body)


def _mm_body(x_ref, w_ref, cnt_ref, h2_ref):
    cnt = jnp.sum(cnt_ref[...], axis=0)
    dinv = lax.rsqrt(cnt + 1.0)
    h = jnp.dot(x_ref[...], w_ref[...], preferred_element_type=jnp.float32)
    h2_ref[...] = h * dinv[:, None]


def _agg_body(h2_ref, srcg_ref, dstg_ref, out_ref,
              acc_sh, idxs_v, idxd_v, buf0, sem0):
    cid = lax.axis_index("c")
    sid = lax.axis_index("s")
    wid = cid * NS + sid
    pltpu.sync_copy(srcg_ref.at[wid], idxs_v)
    pltpu.sync_copy(dstg_ref.at[wid], idxd_v)

    zeros = jnp.zeros((16,), jnp.float32)

    def zero_body(r, carry):
        for c in range(D // 16):
            buf0[r, pl.ds(c * 16, 16)] = zeros
        return carry

    lax.fori_loop(0, CH, zero_body, 0)
    base = sid * STRIPE
    for z in range(STRIPE // CH):
        pltpu.sync_copy(buf0, acc_sh.at[pl.ds(base + z * CH, CH)])
    plsc.subcore_barrier()

    def loop(j, carry):
        pltpu.make_async_copy(h2_ref.at[idxs_v.at[j]], buf0, sem0).start()
        pltpu.make_async_copy(h2_ref.at[idxs_v.at[j]], buf0, sem0).wait()
        pltpu.sync_copy(buf0, acc_sh.at[idxd_v.at[j]], add=True)
        return carry

    lax.fori_loop(0, GPT, loop, 0)
    plsc.subcore_barrier()

    for z in range(STRIPE // CH):
        pltpu.sync_copy(acc_sh.at[pl.ds(base + z * CH, CH)], buf0)
        pltpu.sync_copy(buf0, out_ref.at[cid, pl.ds(base + z * CH, CH)])


_agg_kernel = functools.partial(
    pl.kernel,
    out_type=jax.ShapeDtypeStruct((NC, NP, D), jnp.float32),
    mesh=_mesh,
    scratch_types=[
        pltpu.VMEM_SHARED((NP, D), jnp.float32),
        pltpu.VMEM((GPT, CH), jnp.int32),
        pltpu.VMEM((GPT, CH), jnp.int32),
        pltpu.VMEM((CH, D), jnp.float32),
        pltpu.SemaphoreType.DMA,
    ],
)(_agg_body)


def _fin_body(acc_ref, h2_ref, cnt_ref, b_ref, o_ref):
    cnt = jnp.sum(cnt_ref[...], axis=0)
    dinv = lax.rsqrt(cnt + 1.0)[:, None]
    s = acc_ref[0] + acc_ref[1] + h2_ref[...]
    o_ref[...] = jnp.maximum(s * dinv + b_ref[...], 0.0)


def kernel(x, edge_index, W, b):
    N = x.shape[0]
    E = edge_index.shape[1]
    ei = edge_index.astype(jnp.int32)
    pad = jnp.full((EP - E,), NP - 1, jnp.int32)
    src = jnp.concatenate([ei[0], pad])
    dst = jnp.concatenate([ei[1], pad])
    srcg = src.reshape(NW, GPT, CH)
    dstg = dst.reshape(NW, GPT, CH)
    dst_flat = dst.reshape(NW, EPT)
    x_p = jnp.pad(x, ((0, NP - N), (0, 0)))

    cnt_part = _count_kernel(dst_flat)

    h2 = pl.pallas_call(
        _mm_body,
        grid=(GRID,),
        in_specs=[
            pl.BlockSpec((ROWB, D), lambda i: (i, 0)),
            pl.BlockSpec((D, D), lambda i: (0, 0)),
            pl.BlockSpec((NW, ROWB), lambda i: (0, i)),
        ],
        out_specs=pl.BlockSpec((ROWB, D), lambda i: (i, 0)),
        out_shape=jax.ShapeDtypeStruct((NP, D), jnp.float32),
    )(x_p, W, cnt_part)

    accp = _agg_kernel(h2, srcg, dstg)

    out_p = pl.pallas_call(
        _fin_body,
        grid=(GRID,),
        in_specs=[
            pl.BlockSpec((NC, ROWB, D), lambda i: (0, i, 0)),
            pl.BlockSpec((ROWB, D), lambda i: (i, 0)),
            pl.BlockSpec((NW, ROWB), lambda i: (0, i)),
            pl.BlockSpec((1, D), lambda i: (0, 0)),
        ],
        out_specs=pl.BlockSpec((ROWB, D), lambda i: (i, 0)),
        out_shape=jax.ShapeDtypeStruct((NP, D), jnp.float32),
    )(accp, h2, cnt_part, b.reshape(1, D))

    return out_p[:N]
